# Initial kernel scaffold; baseline (speedup 1.0000x reference)
import jax
import jax.numpy as jnp
from jax.experimental import pallas as pl


def kernel(boxes, classification, rotation, translation):
    raise NotImplementedError("write your pallas kernel here")


# speedup vs baseline: 52.5007x; 52.5007x over previous
import functools

import jax
import jax.numpy as jnp
from jax import lax
from jax.experimental import pallas as pl
from jax.experimental.pallas import tpu as pltpu
from jax.experimental.pallas import tpu_sc as plsc

B = 8
N = 20000
C = 8
MAXD = 100
PAD = 128
L = 16
NEG = float("-inf")
UNROLL = 5
OUTER = N // L // UNROLL
BIG = 2 ** 30
SCORE_T = 0.01
IOU_T = 0.5


def _spf(x):
    return jnp.full((L,), x, jnp.float32)


def _spi(x):
    return jnp.full((L,), x, jnp.int32)


def _sc_body(xT, sT, ftab,
             ocomb, osc, olab,
             sx1, sy1, sx2, sy2, sar, ssc, skidx, sksc,
             m_sc, m_idx, g_idx, g_val, g_sc, g_lab,
             rows, sh_sc, sh_idx, sem):
    core = lax.axis_index("c")
    sub = lax.axis_index("s")
    b = core * 4 + sub // 4
    pair = sub % 4
    c0 = pair * 2
    iota = lax.iota(jnp.int32, L)

    pltpu.sync_copy(xT.at[b, 0], sx1)
    pltpu.sync_copy(xT.at[b, 1], sy1)
    pltpu.sync_copy(xT.at[b, 2], sx2)
    pltpu.sync_copy(xT.at[b, 3], sy2)

    def area_body(j, _):
        for u in range(UNROLL):
            o = (j * UNROLL + u) * L
            w = jnp.maximum(sx2[pl.ds(o, L)] - sx1[pl.ds(o, L)], 0.0)
            h = jnp.maximum(sy2[pl.ds(o, L)] - sy1[pl.ds(o, L)], 0.0)
            sar[pl.ds(o, L)] = w * h
        return 0

    lax.fori_loop(0, OUTER, area_body, 0)

    m0 = _spf(NEG)
    b0 = _spi(0)

    for k in range(2):
        c = c0 + k
        pltpu.sync_copy(sT.at[b, c], ssc)

        def thr_body(j, carry):
            m, bi = carry
            for u in range(UNROLL):
                o = (j * UNROLL + u) * L
                s = ssc[pl.ds(o, L)]
                s = jnp.where(s > SCORE_T, s, NEG)
                ssc[pl.ds(o, L)] = s
                upd = s > m
                m = jnp.where(upd, s, m)
                bi = jnp.where(upd, o + iota, bi)
            return m, bi

        m, bi = lax.fori_loop(0, OUTER, thr_body, (m0, b0))
        M = jnp.max(m)
        I = jnp.min(jnp.where(m == M, bi, BIG))

        def pick(t, carry):
            Mc, Ic = carry
            ok = Mc > NEG
            lane0 = iota == 0
            plsc.store_scatter(skidx, [_spi(k), _spi(t)],
                               _spi(jnp.where(ok, Ic, -1)), mask=lane0)
            plsc.store_scatter(sksc, [_spi(k), _spi(t)],
                               _spf(Mc), mask=lane0)
            iv = _spi(Ic)
            xi1 = plsc.load_gather(sx1, [iv])
            yi1 = plsc.load_gather(sy1, [iv])
            xi2 = plsc.load_gather(sx2, [iv])
            yi2 = plsc.load_gather(sy2, [iv])
            ai = plsc.load_gather(sar, [iv])

            def sup_body(j, carry2):
                m2, bi2 = carry2
                for u in range(UNROLL):
                    o = (j * UNROLL + u) * L
                    s = ssc[pl.ds(o, L)]
                    x1v = sx1[pl.ds(o, L)]
                    y1v = sy1[pl.ds(o, L)]
                    x2v = sx2[pl.ds(o, L)]
                    y2v = sy2[pl.ds(o, L)]
                    av = sar[pl.ds(o, L)]
                    xx1 = jnp.maximum(xi1, x1v)
                    yy1 = jnp.maximum(yi1, y1v)
                    xx2 = jnp.minimum(xi2, x2v)
                    yy2 = jnp.minimum(yi2, y2v)
                    inter = jnp.maximum(xx2 - xx1, 0.0) * jnp.maximum(yy2 - yy1, 0.0)
                    iou = inter / (ai + av - inter + 1e-8)
                    s = jnp.where(iou > IOU_T, NEG, s)
                    ssc[pl.ds(o, L)] = s
                    upd = s > m2
                    m2 = jnp.where(upd, s, m2)
                    bi2 = jnp.where(upd, o + iota, bi2)
                return m2, bi2

            m2, bi2 = lax.fori_loop(0, OUTER, sup_body, (m0, b0))
            M2 = jnp.max(m2)
            I2 = jnp.min(jnp.where(m2 == M2, bi2, BIG))
            return M2, I2

        lax.fori_loop(0, MAXD, pick, (M, I))

    pltpu.sync_copy(sksc, sh_sc.at[b, pl.ds(c0, 2)])
    pltpu.sync_copy(skidx, sh_idx.at[b, pl.ds(c0, 2)])
    plsc.subcore_barrier()

    @pl.when(pair == 0)
    def _merge():
        pltpu.sync_copy(sh_sc.at[b], m_sc)
        pltpu.sync_copy(sh_idx.at[b], m_idx)
        zero = _spi(0)
        for j in range(PAD // L):
            g_idx[pl.ds(j * L, L)] = zero
            g_val[pl.ds(j * L, L)] = zero
        lane_lt = iota < C
        lane_c = jnp.minimum(iota, C - 1)

        def step(t, heads):
            hp = jnp.minimum(heads, PAD - 1)
            hs = plsc.load_gather(m_sc, [lane_c, hp])
            hs = jnp.where(lane_lt & (heads < MAXD), hs, NEG)
            M = jnp.max(hs)
            eq = hs == M
            lane = plsc.all_reduce_ffs(eq)
            lane_v = zero + lane
            is_sel = iota == lane_v
            selt = jnp.min(jnp.where(is_sel, heads, BIG))
            si = plsc.load_gather(m_idx, [lane_v, _spi(selt)])
            sidx = jnp.max(si)
            lane_s = jnp.min(jnp.where(is_sel, iota, BIG))
            ok = M > NEG
            lane0 = iota == 0
            tt = _spi(t)
            plsc.store_scatter(g_sc, [tt], _spf(jnp.where(ok, M, -1.0)), mask=lane0)
            plsc.store_scatter(g_lab, [tt], _spi(jnp.where(ok, lane_s, -1)), mask=lane0)
            plsc.store_scatter(g_idx, [tt],
                               _spi(b * N + jnp.where(ok, sidx, 0)), mask=lane0)
            plsc.store_scatter(g_val, [tt], _spi(jnp.where(ok, 1, 0)), mask=lane0)
            return heads + jnp.where(is_sel & ok, 1, 0)

        lax.fori_loop(0, MAXD, step, _spi(0))

        pltpu.async_copy(ftab.at[g_idx], rows, sem).wait()

        def fix(j, _):
            fo = j * L + iota
            r0 = fo >> 4
            c1 = fo & 15
            v = plsc.load_gather(g_val, [r0]) > 0
            x = plsc.load_gather(rows, [r0, c1])
            plsc.store_scatter(rows, [r0, c1], jnp.where(v, x, -1.0))
            return 0

        lax.fori_loop(0, PAD, fix, 0)

        pltpu.sync_copy(rows, ocomb.at[b])
        pltpu.sync_copy(g_sc, osc.at[b])
        pltpu.sync_copy(g_lab, olab.at[b])


def _make_run(interpret=False):
  return functools.partial(
    pl.kernel,
    interpret=interpret,
    out_type=(
        jax.ShapeDtypeStruct((B, PAD, 16), jnp.float32),
        jax.ShapeDtypeStruct((B, PAD), jnp.float32),
        jax.ShapeDtypeStruct((B, PAD), jnp.int32),
    ),
    mesh=plsc.VectorSubcoreMesh(core_axis_name="c", subcore_axis_name="s",
                                num_cores=2, num_subcores=16),
    compiler_params=pltpu.CompilerParams(
        needs_layout_passes=False, use_tc_tiling_on_sc=False),
    scratch_types=[
        pltpu.VMEM((N,), jnp.float32),
        pltpu.VMEM((N,), jnp.float32),
        pltpu.VMEM((N,), jnp.float32),
        pltpu.VMEM((N,), jnp.float32),
        pltpu.VMEM((N,), jnp.float32),
        pltpu.VMEM((N,), jnp.float32),
        pltpu.VMEM((2, PAD), jnp.int32),
        pltpu.VMEM((2, PAD), jnp.float32),
        pltpu.VMEM((C, PAD), jnp.float32),
        pltpu.VMEM((C, PAD), jnp.int32),
        pltpu.VMEM((PAD,), jnp.int32),
        pltpu.VMEM((PAD,), jnp.int32),
        pltpu.VMEM((PAD,), jnp.float32),
        pltpu.VMEM((PAD,), jnp.int32),
        pltpu.VMEM((PAD, 16), jnp.float32),
        pltpu.VMEM_SHARED((B, C, PAD), jnp.float32),
        pltpu.VMEM_SHARED((B, C, PAD), jnp.int32),
        pltpu.SemaphoreType.DMA,
    ],
  )(_sc_body)


_run = _make_run()


def kernel(boxes, classification, rotation, translation):
    xT = jnp.transpose(boxes, (0, 2, 1))
    sT = jnp.transpose(classification, (0, 2, 1))
    zero1 = jnp.zeros((B, N, 1), jnp.float32)
    ftab = jnp.concatenate(
        [boxes, rotation, zero1, translation, zero1, zero1, zero1, zero1,
         zero1], axis=-1).reshape(B * N, 16)
    comb, sc, lab = _run(xT, sT, ftab)
    return (comb[:, :MAXD, 0:4], sc[:, :MAXD], lab[:, :MAXD],
            comb[:, :MAXD, 4:7], comb[:, :MAXD, 8:11])


# speedup vs baseline: 136.3441x; 2.5970x over previous
import functools

import jax
import jax.numpy as jnp
from jax import lax
from jax.experimental import pallas as pl
from jax.experimental.pallas import tpu as pltpu
from jax.experimental.pallas import tpu_sc as plsc

B = 8
N = 20000
C = 8
MAXD = 100
PAD = 128
L = 16
NEG = float("-inf")
UNROLL = 5
OUTER = N // L // UNROLL
BIG = 2 ** 30
SCORE_T = 0.01
IOU_T = 0.5


def _spf(x):
    return jnp.full((L,), x, jnp.float32)


def _spi(x):
    return jnp.full((L,), x, jnp.int32)


NCH = N // L
NCHP = 1280
NG = NCHP // L


def _sc_body(xT, sT, ftab,
             ocomb, osc, olab,
             sx1, sy1, sx2, sy2, sar, ssc, skidx, sksc,
             chmax, chmax2, kx1, ky1, kx2, ky2, ka,
             m_sc, m_idx, g_idx, g_val, g_sc, g_lab,
             rows, sh_sc, sh_idx, sem):
    core = lax.axis_index("c")
    sub = lax.axis_index("s")
    b = core * 4 + sub // 4
    pair = sub % 4
    c0 = pair * 2
    iota = lax.iota(jnp.int32, L)

    pltpu.sync_copy(xT.at[b, 0], sx1)
    pltpu.sync_copy(xT.at[b, 1], sy1)
    pltpu.sync_copy(xT.at[b, 2], sx2)
    pltpu.sync_copy(xT.at[b, 3], sy2)

    def area_body(j, _):
        for u in range(UNROLL):
            o = (j * UNROLL + u) * L
            w = jnp.maximum(sx2[pl.ds(o, L)] - sx1[pl.ds(o, L)], 0.0)
            h = jnp.maximum(sy2[pl.ds(o, L)] - sy1[pl.ds(o, L)], 0.0)
            sar[pl.ds(o, L)] = w * h
        return 0

    lax.fori_loop(0, OUTER, area_body, 0)

    m0 = _spf(NEG)
    b0 = _spi(0)

    lane0 = iota == 0

    for k in range(2):
        c = c0 + k
        pltpu.sync_copy(sT.at[b, c], ssc)
        chmax[pl.ds(NCH - 2, L)] = m0
        chmax[pl.ds(NCHP - L, L)] = m0
        for q in range(PAD // L):
            kx1[pl.ds(q * L, L)] = _spf(4e9)
            ky1[pl.ds(q * L, L)] = _spf(4e9)
            kx2[pl.ds(q * L, L)] = _spf(-4e9)
            ky2[pl.ds(q * L, L)] = _spf(-4e9)
            ka[pl.ds(q * L, L)] = _spf(0.0)

        def thr_body(j, _):
            for u in range(UNROLL):
                cid = j * UNROLL + u
                o = cid * L
                s = ssc[pl.ds(o, L)]
                s = jnp.where(s > SCORE_T, s, NEG)
                ssc[pl.ds(o, L)] = s
                plsc.store_scatter(chmax, [_spi(cid)], _spf(jnp.max(s)),
                                   mask=lane0)
            return 0

        lax.fori_loop(0, OUTER, thr_body, 0)

        def grp_body(g, _):
            v = chmax[pl.ds(g * L, L)]
            plsc.store_scatter(chmax2, [_spi(g)], _spf(jnp.max(v)),
                               mask=lane0)
            return 0

        lax.fori_loop(0, NG, grp_body, 0)

        mm0 = m0
        for g5 in range(NG // L):
            mm0 = jnp.maximum(mm0, chmax2[pl.ds(g5 * L, L)])
        M0 = jnp.max(mm0)

        def cond(st):
            kc, M = st
            return (kc < MAXD) & (M > NEG)

        def body(st):
            kc, M = st
            vs = []
            gi = jnp.int32(BIG)
            for g5 in range(NG // L):
                v = chmax2[pl.ds(g5 * L, L)]
                vs.append(v)
                gi = jnp.minimum(
                    gi, jnp.min(jnp.where(v == M, g5 * L + iota, BIG)))
            cg = chmax[pl.ds(gi * L, L)]
            ci = jnp.min(jnp.where(cg == M, gi * L + iota, BIG))
            sv = ssc[pl.ds(ci * L, L)]
            j = jnp.min(jnp.where(sv == M, ci * L + iota, BIG))
            jv = _spi(j)
            xj1 = plsc.load_gather(sx1, [jv])
            yj1 = plsc.load_gather(sy1, [jv])
            xj2 = plsc.load_gather(sx2, [jv])
            yj2 = plsc.load_gather(sy2, [jv])
            aj = plsc.load_gather(sar, [jv])

            def vbody(q, accf):
                o = q * L
                x1k = kx1[pl.ds(o, L)]
                y1k = ky1[pl.ds(o, L)]
                x2k = kx2[pl.ds(o, L)]
                y2k = ky2[pl.ds(o, L)]
                ak = ka[pl.ds(o, L)]
                xx1 = jnp.maximum(x1k, xj1)
                yy1 = jnp.maximum(y1k, yj1)
                xx2 = jnp.minimum(x2k, xj2)
                yy2 = jnp.minimum(y2k, yj2)
                inter = jnp.maximum(xx2 - xx1, 0.0) * jnp.maximum(yy2 - yy1, 0.0)
                iou = inter / (ak + aj - inter + 1e-8)
                return jnp.maximum(accf, iou)

            nv = (kc + L - 1) // L
            accf = lax.fori_loop(0, nv, vbody, jnp.zeros((L,), jnp.float32))
            sup = jnp.max(accf) > IOU_T

            @pl.when(jnp.logical_not(sup))
            def _():
                kv = _spi(kc)
                plsc.store_scatter(skidx, [_spi(k), kv], jv, mask=lane0)
                plsc.store_scatter(sksc, [_spi(k), kv], _spf(M), mask=lane0)
                plsc.store_scatter(kx1, [kv], xj1, mask=lane0)
                plsc.store_scatter(ky1, [kv], yj1, mask=lane0)
                plsc.store_scatter(kx2, [kv], xj2, mask=lane0)
                plsc.store_scatter(ky2, [kv], yj2, mask=lane0)
                plsc.store_scatter(ka, [kv], aj, mask=lane0)

            plsc.store_scatter(ssc, [jv], _spf(NEG), mask=lane0)
            sv2 = jnp.where(ci * L + iota == j, NEG, sv)
            newcm = jnp.max(sv2)
            plsc.store_scatter(chmax, [_spi(ci)], _spf(newcm), mask=lane0)
            cg2 = jnp.where(gi * L + iota == ci, newcm, cg)
            newgm = jnp.max(cg2)
            plsc.store_scatter(chmax2, [_spi(gi)], _spf(newgm), mask=lane0)
            mm = m0
            for g5 in range(NG // L):
                vg = jnp.where(g5 * L + iota == gi, newgm, vs[g5])
                mm = jnp.maximum(mm, vg)
            M2 = jnp.max(mm)
            kc2 = kc + jnp.where(sup, 0, 1)
            return (kc2, M2)

        kcF, _ = lax.while_loop(cond, body, (jnp.int32(0), M0))

        def fill(t, _):
            plsc.store_scatter(skidx, [_spi(k), _spi(t)], _spi(-1), mask=lane0)
            plsc.store_scatter(sksc, [_spi(k), _spi(t)], _spf(NEG), mask=lane0)
            return 0

        lax.fori_loop(kcF, MAXD, fill, 0)

    pltpu.sync_copy(sksc, sh_sc.at[b, pl.ds(c0, 2)])
    pltpu.sync_copy(skidx, sh_idx.at[b, pl.ds(c0, 2)])
    plsc.subcore_barrier()

    @pl.when(pair == 0)
    def _merge():
        pltpu.sync_copy(sh_sc.at[b], m_sc)
        pltpu.sync_copy(sh_idx.at[b], m_idx)
        zero = _spi(0)
        for j in range(PAD // L):
            g_idx[pl.ds(j * L, L)] = zero
            g_val[pl.ds(j * L, L)] = zero
        lane_lt = iota < C
        lane_c = jnp.minimum(iota, C - 1)

        def step(t, heads):
            hp = jnp.minimum(heads, PAD - 1)
            hs = plsc.load_gather(m_sc, [lane_c, hp])
            hs = jnp.where(lane_lt & (heads < MAXD), hs, NEG)
            M = jnp.max(hs)
            eq = hs == M
            lane = plsc.all_reduce_ffs(eq)
            lane_v = zero + lane
            is_sel = iota == lane_v
            selt = jnp.min(jnp.where(is_sel, heads, BIG))
            si = plsc.load_gather(m_idx, [lane_v, _spi(selt)])
            sidx = jnp.max(si)
            lane_s = jnp.min(jnp.where(is_sel, iota, BIG))
            ok = M > NEG
            lane0 = iota == 0
            tt = _spi(t)
            plsc.store_scatter(g_sc, [tt], _spf(jnp.where(ok, M, -1.0)), mask=lane0)
            plsc.store_scatter(g_lab, [tt], _spi(jnp.where(ok, lane_s, -1)), mask=lane0)
            plsc.store_scatter(g_idx, [tt],
                               _spi(b * N + jnp.where(ok, sidx, 0)), mask=lane0)
            plsc.store_scatter(g_val, [tt], _spi(jnp.where(ok, 1, 0)), mask=lane0)
            return heads + jnp.where(is_sel & ok, 1, 0)

        lax.fori_loop(0, MAXD, step, _spi(0))

        pltpu.async_copy(ftab.at[g_idx], rows, sem).wait()

        def fix(j, _):
            fo = j * L + iota
            r0 = fo >> 4
            c1 = fo & 15
            v = plsc.load_gather(g_val, [r0]) > 0
            x = plsc.load_gather(rows, [r0, c1])
            plsc.store_scatter(rows, [r0, c1], jnp.where(v, x, -1.0))
            return 0

        lax.fori_loop(0, PAD, fix, 0)

        pltpu.sync_copy(rows, ocomb.at[b])
        pltpu.sync_copy(g_sc, osc.at[b])
        pltpu.sync_copy(g_lab, olab.at[b])


def _make_run(interpret=False):
  return functools.partial(
    pl.kernel,
    interpret=interpret,
    out_type=(
        jax.ShapeDtypeStruct((B, PAD, 16), jnp.float32),
        jax.ShapeDtypeStruct((B, PAD), jnp.float32),
        jax.ShapeDtypeStruct((B, PAD), jnp.int32),
    ),
    mesh=plsc.VectorSubcoreMesh(core_axis_name="c", subcore_axis_name="s",
                                num_cores=2, num_subcores=16),
    compiler_params=pltpu.CompilerParams(
        needs_layout_passes=False, use_tc_tiling_on_sc=False),
    scratch_types=[
        pltpu.VMEM((N,), jnp.float32),
        pltpu.VMEM((N,), jnp.float32),
        pltpu.VMEM((N,), jnp.float32),
        pltpu.VMEM((N,), jnp.float32),
        pltpu.VMEM((N,), jnp.float32),
        pltpu.VMEM((N,), jnp.float32),
        pltpu.VMEM((2, PAD), jnp.int32),
        pltpu.VMEM((2, PAD), jnp.float32),
        pltpu.VMEM((NCHP,), jnp.float32),
        pltpu.VMEM((NG,), jnp.float32),
        pltpu.VMEM((PAD,), jnp.float32),
        pltpu.VMEM((PAD,), jnp.float32),
        pltpu.VMEM((PAD,), jnp.float32),
        pltpu.VMEM((PAD,), jnp.float32),
        pltpu.VMEM((PAD,), jnp.float32),
        pltpu.VMEM((C, PAD), jnp.float32),
        pltpu.VMEM((C, PAD), jnp.int32),
        pltpu.VMEM((PAD,), jnp.int32),
        pltpu.VMEM((PAD,), jnp.int32),
        pltpu.VMEM((PAD,), jnp.float32),
        pltpu.VMEM((PAD,), jnp.int32),
        pltpu.VMEM((PAD, 16), jnp.float32),
        pltpu.VMEM_SHARED((B, C, PAD), jnp.float32),
        pltpu.VMEM_SHARED((B, C, PAD), jnp.int32),
        pltpu.SemaphoreType.DMA,
    ],
  )(_sc_body)


_run = _make_run()


def kernel(boxes, classification, rotation, translation):
    xT = jnp.transpose(boxes, (0, 2, 1))
    sT = jnp.transpose(classification, (0, 2, 1))
    zero1 = jnp.zeros((B, N, 1), jnp.float32)
    ftab = jnp.concatenate(
        [boxes, rotation, zero1, translation, zero1, zero1, zero1, zero1,
         zero1], axis=-1).reshape(B * N, 16)
    comb, sc, lab = _run(xT, sT, ftab)
    return (comb[:, :MAXD, 0:4], sc[:, :MAXD], lab[:, :MAXD],
            comb[:, :MAXD, 4:7], comb[:, :MAXD, 8:11])
